# Initial kernel scaffold; baseline (speedup 1.0000x reference)
import jax
import jax.numpy as jnp
from jax.experimental import pallas as pl


def kernel(x, ln1_g, ln1_b, Wq, bq, Wk, bk, Wv, bv, Wo, bo, ln2_g, ln2_b, Wg, W1, B1, W2, B2):
    raise NotImplementedError("write your pallas kernel here")


# speedup vs baseline: 1.4303x; 1.4303x over previous
import functools

import jax
import jax.numpy as jnp
from jax import lax
from jax.experimental import pallas as pl
from jax.experimental.pallas import tpu as pltpu

try:
    from jax.experimental.pallas import tpu_sc as plsc
    _HAS_SC = True
except ImportError:
    _HAS_SC = False

S, B, D, H, FFN, E = 2048, 1, 2048, 16, 8192, 8
DH = D // H
C = 2 * S * B // E
EP = 128
NEG = -1e30

f32 = jnp.float32
i32 = jnp.int32


def _ln_body(x_ref, g_ref, b_ref, h_ref):
    xx = x_ref[...]
    mu = jnp.mean(xx, axis=1, keepdims=True)
    var = jnp.mean((xx - mu) ** 2, axis=1, keepdims=True)
    h_ref[...] = (xx - mu) / jnp.sqrt(var + 1e-5) * g_ref[...] + b_ref[...]


def _ln(x, g, b, bm=512):
    return pl.pallas_call(
        _ln_body,
        grid=(S // bm,),
        in_specs=[pl.BlockSpec((bm, D), lambda i: (i, 0)),
                  pl.BlockSpec((1, D), lambda i: (0, 0)),
                  pl.BlockSpec((1, D), lambda i: (0, 0))],
        out_specs=pl.BlockSpec((bm, D), lambda i: (i, 0)),
        out_shape=jax.ShapeDtypeStruct((S, D), f32),
    )(x, g.reshape(1, D), b.reshape(1, D))


def _mm_body(a_ref, w_ref, b_ref, r_ref, o_ref):
    o = jnp.dot(a_ref[...], w_ref[...], preferred_element_type=f32)
    o = o + b_ref[...]
    if r_ref is not None:
        o = r_ref[...] + o
    o_ref[...] = o


def _mm(a, w, bias, res=None, bn=512):
    nn = D // bn
    in_specs = [pl.BlockSpec((S, D), lambda j: (0, 0)),
                pl.BlockSpec((D, bn), lambda j: (0, j)),
                pl.BlockSpec((1, bn), lambda j: (0, j))]
    args = [a, w, bias.reshape(1, D)]
    if res is not None:
        in_specs.append(pl.BlockSpec((S, bn), lambda j: (0, j)))
        args.append(res)
        body = _mm_body
    else:
        def body(a_ref, w_ref, b_ref, o_ref):
            _mm_body(a_ref, w_ref, b_ref, None, o_ref)
    return pl.pallas_call(
        body,
        grid=(nn,),
        in_specs=in_specs,
        out_specs=pl.BlockSpec((S, bn), lambda j: (0, j)),
        out_shape=jax.ShapeDtypeStruct((S, D), f32),
    )(*args)


def _attn_body(q_ref, k_ref, v_ref, o_ref, *, bq, kc):
    qq = q_ref[...]
    m = jnp.full((bq, 1), -jnp.inf, f32)
    l = jnp.zeros((bq, 1), f32)
    o = jnp.zeros((bq, DH), f32)
    for c in range(S // kc):
        kk = k_ref[c * kc:(c + 1) * kc, :]
        vv = v_ref[c * kc:(c + 1) * kc, :]
        s = lax.dot_general(qq, kk, (((1,), (1,)), ((), ())),
                            preferred_element_type=f32) / jnp.sqrt(jnp.float32(DH))
        mc = jnp.max(s, axis=1, keepdims=True)
        mn = jnp.maximum(m, mc)
        p = jnp.exp(s - mn)
        corr = jnp.exp(m - mn)
        l = l * corr + jnp.sum(p, axis=1, keepdims=True)
        o = o * corr + lax.dot_general(p, vv, (((1,), (0,)), ((), ())),
                                       preferred_element_type=f32)
        m = mn
    o_ref[...] = o / l


def _attn(q, k, v, bq=512, kc=1024):
    nq = S // bq
    return pl.pallas_call(
        functools.partial(_attn_body, bq=bq, kc=kc),
        grid=(H, nq),
        in_specs=[
            pl.BlockSpec((bq, DH), lambda h, i: (i, h)),
            pl.BlockSpec((S, DH), lambda h, i: (0, h)),
            pl.BlockSpec((S, DH), lambda h, i: (0, h)),
        ],
        out_specs=pl.BlockSpec((bq, DH), lambda h, i: (i, h)),
        out_shape=jax.ShapeDtypeStruct((S, D), f32),
    )(q, k, v)


def _logits_body(h_ref, wg_ref, lg_ref):
    lg = jnp.dot(h_ref[...], wg_ref[...], preferred_element_type=f32)
    colpad = lax.broadcasted_iota(i32, lg.shape, 1) >= E
    lg_ref[...] = jnp.where(colpad, NEG, lg)


def _logits(h2, Wg_pad):
    return pl.pallas_call(
        _logits_body,
        in_specs=[pl.BlockSpec((S, D), lambda: (0, 0)),
                  pl.BlockSpec((D, EP), lambda: (0, 0))],
        out_specs=pl.BlockSpec((S, EP), lambda: (0, 0)),
        out_shape=jax.ShapeDtypeStruct((S, EP), f32),
    )(h2, Wg_pad)


def _gate_body(lg_ref, g1_ref, g2_ref, i1_ref, i2_ref, s2t_ref, laux_ref,
               m1_s, m2_s, l1_s, l2_s):
    T = S
    lg = lg_ref[...]
    mx = jnp.max(lg, axis=1, keepdims=True)
    ex = jnp.exp(lg - mx)
    sm = ex / jnp.sum(ex, axis=1, keepdims=True)
    col = lax.broadcasted_iota(i32, (T, EP), 1)
    e1 = jnp.min(jnp.where(lg == mx, col, EP), axis=1)
    m1 = (col == e1[:, None]).astype(f32)
    lg2 = jnp.where(m1 > 0, -1e9, lg)
    mx2 = jnp.max(lg2, axis=1, keepdims=True)
    e2 = jnp.min(jnp.where(lg2 == mx2, col, EP), axis=1)
    m2 = (col == e2[:, None]).astype(f32)
    g1 = jnp.sum(sm * m1, axis=1)
    g2 = jnp.sum(sm * m2, axis=1)

    me = jnp.mean(sm, axis=0)
    ce = jnp.mean(m1, axis=0)
    laux_ref[...] = (jnp.sum(me * ce) * E).reshape(1, 1)

    m1_s[...] = m1
    m2_s[...] = m2
    CH = 256
    r = lax.broadcasted_iota(i32, (CH, CH), 0)
    c = lax.broadcasted_iota(i32, (CH, CH), 1)
    tri = (c < r).astype(f32)

    def cum_step(cb, carry):
        c1, c2 = carry
        a1 = m1_s[pl.ds(cb * CH, CH), :]
        a2 = m2_s[pl.ds(cb * CH, CH), :]
        l1_s[pl.ds(cb * CH, CH), :] = jnp.dot(tri, a1, preferred_element_type=f32) + c1
        l2_s[pl.ds(cb * CH, CH), :] = jnp.dot(tri, a2, preferred_element_type=f32) + c2
        return (c1 + jnp.sum(a1, axis=0, keepdims=True),
                c2 + jnp.sum(a2, axis=0, keepdims=True))

    zero = jnp.zeros((1, EP), f32)
    cnt1, _ = lax.fori_loop(0, T // CH, cum_step, (zero, zero))
    loc1 = l1_s[...]
    loc2 = l2_s[...] + cnt1

    m1k = m1 * (loc1 < C).astype(f32)
    m2k = m2 * (loc2 < C).astype(f32)
    p1 = jnp.sum(loc1 * m1k, axis=1).astype(i32)
    p2 = jnp.sum(loc2 * m2k, axis=1).astype(i32)
    k1 = jnp.sum(m1k, axis=1)
    k2 = jnp.sum(m2k, axis=1)
    g1 = g1 * k1
    g2 = g2 * k2
    den = jnp.maximum(g1 + g2, 1e-9)
    g1_ref[...] = (g1 / den)[:, None]
    g2_ref[...] = (g2 / den)[:, None]

    s1 = e1 * C + p1
    s2 = e2 * C + p2
    s1m = jnp.where(k1 > 0, s1, -1)
    s2m = jnp.where(k2 > 0, s2, -1)
    i1_ref[...] = jnp.where(k1 > 0, s1, 0)[:, None]
    i2_ref[...] = jnp.where(k2 > 0, s2, 0)[:, None]

    SB = 512
    tok1 = (lax.broadcasted_iota(i32, (T, SB), 0) + 1)

    def inv_step(sb, _):
        slot = lax.broadcasted_iota(i32, (T, SB), 1) + sb * SB
        val = (jnp.where(s1m[:, None] == slot, tok1, 0)
               + jnp.where(s2m[:, None] == slot, tok1, 0))
        st = jnp.sum(val, axis=0)
        s2t_ref[pl.ds(sb, 1), :] = jnp.maximum(st - 1, 0)[None, :]
        return 0

    lax.fori_loop(0, (E * C) // SB, inv_step, 0)


def _gate(logits_pad):
    return pl.pallas_call(
        _gate_body,
        out_shape=[
            jax.ShapeDtypeStruct((S, 1), f32),
            jax.ShapeDtypeStruct((S, 1), f32),
            jax.ShapeDtypeStruct((S, 1), i32),
            jax.ShapeDtypeStruct((S, 1), i32),
            jax.ShapeDtypeStruct((E * C // 512, 512), i32),
            jax.ShapeDtypeStruct((1, 1), f32),
        ],
        scratch_shapes=[pltpu.VMEM((S, EP), f32)] * 4,
    )(logits_pad)


def _sc_gather(table, idx):
    V, Dd = table.shape
    (Bn,) = idx.shape
    return _make_sc_gather(V, Bn, Dd)(table, idx)


@functools.lru_cache(maxsize=None)
def _make_sc_gather(V, Bn, Dd):
    info = plsc.get_sparse_core_info()
    NC, NS = info.num_cores, info.num_subcores
    NW = NC * NS
    bpw = Bn // NW
    CH = 16 if bpw % 16 == 0 else 8
    nch = bpw // CH
    mesh = plsc.VectorSubcoreMesh(core_axis_name="c", subcore_axis_name="s")

    @functools.partial(
        pl.kernel, mesh=mesh,
        out_type=jax.ShapeDtypeStruct((Bn, Dd), f32),
        scratch_types=[
            pltpu.VMEM((bpw,), i32),
            pltpu.VMEM((CH, Dd), f32),
            pltpu.VMEM((CH, Dd), f32),
            pltpu.SemaphoreType.DMA,
            pltpu.SemaphoreType.DMA,
        ],
    )
    def k(table_hbm, idx_hbm, out_hbm, idx_v, buf0, buf1, sem0, sem1):
        wid = lax.axis_index("s") * NC + lax.axis_index("c")
        base = wid * bpw
        pltpu.sync_copy(idx_hbm.at[pl.ds(base, bpw)], idx_v)
        bufs = (buf0, buf1)
        sems = (sem0, sem1)
        cps = [None, None]
        cps[0] = pltpu.async_copy(table_hbm.at[idx_v.at[pl.ds(0, CH)]],
                                  bufs[0], sems[0])
        for c0 in range(nch):
            cur = c0 % 2
            nxt = (c0 + 1) % 2
            if c0 + 1 < nch:
                cps[nxt] = pltpu.async_copy(
                    table_hbm.at[idx_v.at[pl.ds((c0 + 1) * CH, CH)]],
                    bufs[nxt], sems[nxt])
            cps[cur].wait()
            pltpu.sync_copy(bufs[cur], out_hbm.at[pl.ds(base + c0 * CH, CH)])

    return k


def _ffn_body(de_ref, w1_ref, b1_ref, w2_ref, b2_ref, o_ref):
    f = pl.program_id(1)
    de = de_ref[0]
    hh = jnp.maximum(jnp.dot(de, w1_ref[0], preferred_element_type=f32)
                     + b1_ref[0], 0.0)
    contrib = jnp.dot(hh, w2_ref[0], preferred_element_type=f32)

    @pl.when(f == 0)
    def _():
        o_ref[0] = contrib + b2_ref[0]

    @pl.when(f > 0)
    def _():
        o_ref[0] = o_ref[0] + contrib


def _ffn(de3, W1, B1, W2, B2, bf=1024):
    nf = FFN // bf
    return pl.pallas_call(
        _ffn_body,
        grid=(E, nf),
        in_specs=[
            pl.BlockSpec((1, C, D), lambda e, f: (e, 0, 0)),
            pl.BlockSpec((1, D, bf), lambda e, f: (e, 0, f)),
            pl.BlockSpec((1, 1, bf), lambda e, f: (e, 0, f)),
            pl.BlockSpec((1, bf, D), lambda e, f: (e, f, 0)),
            pl.BlockSpec((1, 1, D), lambda e, f: (e, 0, 0)),
        ],
        out_specs=pl.BlockSpec((1, C, D), lambda e, f: (e, 0, 0)),
        out_shape=jax.ShapeDtypeStruct((E, C, D), f32),
    )(de3, W1, B1.reshape(E, 1, FFN), W2, B2.reshape(E, 1, D))


def _combine_body(x_ref, r1_ref, r2_ref, g1_ref, g2_ref, o_ref):
    o_ref[...] = (x_ref[...] + g1_ref[...] * r1_ref[...]
                  + g2_ref[...] * r2_ref[...])


def _combine(x2, rows1, rows2, g1, g2, bm=512):
    nm = S // bm
    return pl.pallas_call(
        _combine_body,
        grid=(nm,),
        in_specs=[
            pl.BlockSpec((bm, D), lambda i: (i, 0)),
            pl.BlockSpec((bm, D), lambda i: (i, 0)),
            pl.BlockSpec((bm, D), lambda i: (i, 0)),
            pl.BlockSpec((bm, 1), lambda i: (i, 0)),
            pl.BlockSpec((bm, 1), lambda i: (i, 0)),
        ],
        out_specs=pl.BlockSpec((bm, D), lambda i: (i, 0)),
        out_shape=jax.ShapeDtypeStruct((S, D), f32),
    )(x2, rows1, rows2, g1, g2)


def kernel(x, ln1_g, ln1_b, Wq, bq, Wk, bk, Wv, bv, Wo, bo, ln2_g, ln2_b,
           Wg, W1, B1, W2, B2):
    x2d = x.reshape(S, D)
    h = _ln(x2d, ln1_g, ln1_b)
    q = _mm(h, Wq, bq)
    k = _mm(h, Wk, bk)
    v = _mm(h, Wv, bv)
    ao = _attn(q, k, v)
    x2 = _mm(ao, Wo, bo, res=x2d)
    h2 = _ln(x2, ln2_g, ln2_b)
    Wg_pad = jnp.zeros((D, EP), f32).at[:, :E].set(Wg)
    logits = _logits(h2, Wg_pad)
    g1, g2, i1, i2, s2t, laux = _gate(logits)
    de = _sc_gather(h2, s2t.reshape(E * C))
    eo = _ffn(de.reshape(E, C, D), W1, B1, W2, B2)
    eo2 = eo.reshape(E * C, D)
    rows = _sc_gather(eo2, jnp.concatenate([i1.reshape(S), i2.reshape(S)]))
    out = _combine(x2, rows[:S], rows[S:], g1, g2)
    return out.reshape(S, B, D), laux[0, 0]


# speedup vs baseline: 1.4354x; 1.0036x over previous
import functools

import jax
import jax.numpy as jnp
from jax import lax
from jax.experimental import pallas as pl
from jax.experimental.pallas import tpu as pltpu

try:
    from jax.experimental.pallas import tpu_sc as plsc
    _HAS_SC = True
except ImportError:
    _HAS_SC = False

S, B, D, H, FFN, E = 2048, 1, 2048, 16, 8192, 8
DH = D // H
C = 2 * S * B // E
EP = 128
NEG = -1e30

f32 = jnp.float32
i32 = jnp.int32


def _ln_body(x_ref, g_ref, b_ref, h_ref):
    xx = x_ref[...]
    mu = jnp.mean(xx, axis=1, keepdims=True)
    var = jnp.mean((xx - mu) ** 2, axis=1, keepdims=True)
    h_ref[...] = (xx - mu) / jnp.sqrt(var + 1e-5) * g_ref[...] + b_ref[...]


def _ln(x, g, b, bm=512):
    return pl.pallas_call(
        _ln_body,
        grid=(S // bm,),
        in_specs=[pl.BlockSpec((bm, D), lambda i: (i, 0)),
                  pl.BlockSpec((1, D), lambda i: (0, 0)),
                  pl.BlockSpec((1, D), lambda i: (0, 0))],
        out_specs=pl.BlockSpec((bm, D), lambda i: (i, 0)),
        out_shape=jax.ShapeDtypeStruct((S, D), f32),
    )(x, g.reshape(1, D), b.reshape(1, D))


def _mm_body(a_ref, w_ref, b_ref, r_ref, o_ref):
    o = jnp.dot(a_ref[...], w_ref[...], preferred_element_type=f32)
    o = o + b_ref[...]
    if r_ref is not None:
        o = r_ref[...] + o
    o_ref[...] = o


def _mm(a, w, bias, res=None, bn=512):
    nn = D // bn
    in_specs = [pl.BlockSpec((S, D), lambda j: (0, 0)),
                pl.BlockSpec((D, bn), lambda j: (0, j)),
                pl.BlockSpec((1, bn), lambda j: (0, j))]
    args = [a, w, bias.reshape(1, D)]
    if res is not None:
        in_specs.append(pl.BlockSpec((S, bn), lambda j: (0, j)))
        args.append(res)
        body = _mm_body
    else:
        def body(a_ref, w_ref, b_ref, o_ref):
            _mm_body(a_ref, w_ref, b_ref, None, o_ref)
    return pl.pallas_call(
        body,
        grid=(nn,),
        in_specs=in_specs,
        out_specs=pl.BlockSpec((S, bn), lambda j: (0, j)),
        out_shape=jax.ShapeDtypeStruct((S, D), f32),
    )(*args)


def _attn_body(q_ref, k_ref, v_ref, o_ref, *, bq, kc):
    qq = q_ref[...]
    m = jnp.full((bq, 1), -jnp.inf, f32)
    l = jnp.zeros((bq, 1), f32)
    o = jnp.zeros((bq, DH), f32)
    for c in range(S // kc):
        kk = k_ref[c * kc:(c + 1) * kc, :]
        vv = v_ref[c * kc:(c + 1) * kc, :]
        s = lax.dot_general(qq, kk, (((1,), (1,)), ((), ())),
                            preferred_element_type=f32) / jnp.sqrt(jnp.float32(DH))
        mc = jnp.max(s, axis=1, keepdims=True)
        mn = jnp.maximum(m, mc)
        p = jnp.exp(s - mn)
        corr = jnp.exp(m - mn)
        l = l * corr + jnp.sum(p, axis=1, keepdims=True)
        o = o * corr + lax.dot_general(p, vv, (((1,), (0,)), ((), ())),
                                       preferred_element_type=f32)
        m = mn
    o_ref[...] = o / l


def _attn(q, k, v, bq=512, kc=1024):
    nq = S // bq
    return pl.pallas_call(
        functools.partial(_attn_body, bq=bq, kc=kc),
        grid=(H, nq),
        in_specs=[
            pl.BlockSpec((bq, DH), lambda h, i: (i, h)),
            pl.BlockSpec((S, DH), lambda h, i: (0, h)),
            pl.BlockSpec((S, DH), lambda h, i: (0, h)),
        ],
        out_specs=pl.BlockSpec((bq, DH), lambda h, i: (i, h)),
        out_shape=jax.ShapeDtypeStruct((S, D), f32),
    )(q, k, v)


def _logits_body(h_ref, wg_ref, lg_ref):
    lg = jnp.dot(h_ref[...], wg_ref[...], preferred_element_type=f32)
    colpad = lax.broadcasted_iota(i32, lg.shape, 1) >= E
    lg_ref[...] = jnp.where(colpad, NEG, lg)


def _logits(h2, Wg_pad):
    return pl.pallas_call(
        _logits_body,
        in_specs=[pl.BlockSpec((S, D), lambda: (0, 0)),
                  pl.BlockSpec((D, EP), lambda: (0, 0))],
        out_specs=pl.BlockSpec((S, EP), lambda: (0, 0)),
        out_shape=jax.ShapeDtypeStruct((S, EP), f32),
    )(h2, Wg_pad)


def _gate_body(lg_ref, g1_ref, g2_ref, i1_ref, i2_ref, s2t_ref, laux_ref,
               m1_s, m2_s, l1_s, l2_s):
    T = S
    lg = lg_ref[...]
    mx = jnp.max(lg, axis=1, keepdims=True)
    ex = jnp.exp(lg - mx)
    sm = ex / jnp.sum(ex, axis=1, keepdims=True)
    col = lax.broadcasted_iota(i32, (T, EP), 1)
    e1 = jnp.min(jnp.where(lg == mx, col, EP), axis=1)
    m1 = (col == e1[:, None]).astype(f32)
    lg2 = jnp.where(m1 > 0, -1e9, lg)
    mx2 = jnp.max(lg2, axis=1, keepdims=True)
    e2 = jnp.min(jnp.where(lg2 == mx2, col, EP), axis=1)
    m2 = (col == e2[:, None]).astype(f32)
    g1 = jnp.sum(sm * m1, axis=1)
    g2 = jnp.sum(sm * m2, axis=1)

    me = jnp.mean(sm, axis=0)
    ce = jnp.mean(m1, axis=0)
    laux_ref[...] = (jnp.sum(me * ce) * E).reshape(1, 1)

    m1_s[...] = m1
    m2_s[...] = m2
    CH = 256
    r = lax.broadcasted_iota(i32, (CH, CH), 0)
    c = lax.broadcasted_iota(i32, (CH, CH), 1)
    tri = (c < r).astype(f32)

    def cum_step(cb, carry):
        c1, c2 = carry
        a1 = m1_s[pl.ds(cb * CH, CH), :]
        a2 = m2_s[pl.ds(cb * CH, CH), :]
        l1_s[pl.ds(cb * CH, CH), :] = jnp.dot(tri, a1, preferred_element_type=f32) + c1
        l2_s[pl.ds(cb * CH, CH), :] = jnp.dot(tri, a2, preferred_element_type=f32) + c2
        return (c1 + jnp.sum(a1, axis=0, keepdims=True),
                c2 + jnp.sum(a2, axis=0, keepdims=True))

    zero = jnp.zeros((1, EP), f32)
    cnt1, _ = lax.fori_loop(0, T // CH, cum_step, (zero, zero))
    loc1 = l1_s[...]
    loc2 = l2_s[...] + cnt1

    m1k = m1 * (loc1 < C).astype(f32)
    m2k = m2 * (loc2 < C).astype(f32)
    p1 = jnp.sum(loc1 * m1k, axis=1).astype(i32)
    p2 = jnp.sum(loc2 * m2k, axis=1).astype(i32)
    k1 = jnp.sum(m1k, axis=1)
    k2 = jnp.sum(m2k, axis=1)
    g1 = g1 * k1
    g2 = g2 * k2
    den = jnp.maximum(g1 + g2, 1e-9)
    g1_ref[...] = (g1 / den)[:, None]
    g2_ref[...] = (g2 / den)[:, None]

    s1 = e1 * C + p1
    s2 = e2 * C + p2
    s1m = jnp.where(k1 > 0, s1, -1)
    s2m = jnp.where(k2 > 0, s2, -1)
    i1_ref[...] = jnp.where(k1 > 0, s1, 0)[:, None]
    i2_ref[...] = jnp.where(k2 > 0, s2, 0)[:, None]

    SB = 512
    tok1 = (lax.broadcasted_iota(i32, (T, SB), 0) + 1)

    def inv_step(sb, _):
        slot = lax.broadcasted_iota(i32, (T, SB), 1) + sb * SB
        val = (jnp.where(s1m[:, None] == slot, tok1, 0)
               + jnp.where(s2m[:, None] == slot, tok1, 0))
        st = jnp.sum(val, axis=0)
        s2t_ref[pl.ds(sb, 1), :] = jnp.maximum(st - 1, 0)[None, :]
        return 0

    lax.fori_loop(0, (E * C) // SB, inv_step, 0)


def _gate(logits_pad):
    return pl.pallas_call(
        _gate_body,
        out_shape=[
            jax.ShapeDtypeStruct((S, 1), f32),
            jax.ShapeDtypeStruct((S, 1), f32),
            jax.ShapeDtypeStruct((S, 1), i32),
            jax.ShapeDtypeStruct((S, 1), i32),
            jax.ShapeDtypeStruct((E * C // 512, 512), i32),
            jax.ShapeDtypeStruct((1, 1), f32),
        ],
        scratch_shapes=[pltpu.VMEM((S, EP), f32)] * 4,
    )(logits_pad)


def _sc_gather(table, idx):
    V, Dd = table.shape
    (Bn,) = idx.shape
    return _make_sc_gather(V, Bn, Dd)(table, idx)


@functools.lru_cache(maxsize=None)
def _make_sc_gather(V, Bn, Dd):
    info = plsc.get_sparse_core_info()
    NC, NS = info.num_cores, info.num_subcores
    NW = NC * NS
    bpw = Bn // NW
    CH = 16 if bpw % 16 == 0 else 8
    nch = bpw // CH
    mesh = plsc.VectorSubcoreMesh(core_axis_name="c", subcore_axis_name="s")

    @functools.partial(
        pl.kernel, mesh=mesh,
        out_type=jax.ShapeDtypeStruct((Bn, Dd), f32),
        scratch_types=[
            pltpu.VMEM((bpw,), i32),
            pltpu.VMEM((CH, Dd), f32),
            pltpu.VMEM((CH, Dd), f32),
            pltpu.SemaphoreType.DMA,
            pltpu.SemaphoreType.DMA,
        ],
    )
    def k(table_hbm, idx_hbm, out_hbm, idx_v, buf0, buf1, sem0, sem1):
        wid = lax.axis_index("s") * NC + lax.axis_index("c")
        base = wid * bpw
        pltpu.sync_copy(idx_hbm.at[pl.ds(base, bpw)], idx_v)
        bufs = (buf0, buf1)
        sems = (sem0, sem1)
        cps = [None, None]
        cps[0] = pltpu.async_copy(table_hbm.at[idx_v.at[pl.ds(0, CH)]],
                                  bufs[0], sems[0])
        for c0 in range(nch):
            cur = c0 % 2
            nxt = (c0 + 1) % 2
            if c0 + 1 < nch:
                cps[nxt] = pltpu.async_copy(
                    table_hbm.at[idx_v.at[pl.ds((c0 + 1) * CH, CH)]],
                    bufs[nxt], sems[nxt])
            cps[cur].wait()
            pltpu.sync_copy(bufs[cur], out_hbm.at[pl.ds(base + c0 * CH, CH)])

    return k


def _ffn_body(de_ref, w1_ref, b1_ref, w2_ref, b2_ref, o_ref):
    f = pl.program_id(1)
    de = de_ref[0].astype(jnp.bfloat16)
    hh = jnp.maximum(jnp.dot(de, w1_ref[0].astype(jnp.bfloat16),
                             preferred_element_type=f32) + b1_ref[0], 0.0)
    contrib = jnp.dot(hh.astype(jnp.bfloat16), w2_ref[0].astype(jnp.bfloat16),
                      preferred_element_type=f32)

    @pl.when(f == 0)
    def _():
        o_ref[0] = contrib + b2_ref[0]

    @pl.when(f > 0)
    def _():
        o_ref[0] = o_ref[0] + contrib


def _ffn(de3, W1, B1, W2, B2, bf=1024):
    nf = FFN // bf
    return pl.pallas_call(
        _ffn_body,
        grid=(E, nf),
        in_specs=[
            pl.BlockSpec((1, C, D), lambda e, f: (e, 0, 0)),
            pl.BlockSpec((1, D, bf), lambda e, f: (e, 0, f)),
            pl.BlockSpec((1, 1, bf), lambda e, f: (e, 0, f)),
            pl.BlockSpec((1, bf, D), lambda e, f: (e, f, 0)),
            pl.BlockSpec((1, 1, D), lambda e, f: (e, 0, 0)),
        ],
        out_specs=pl.BlockSpec((1, C, D), lambda e, f: (e, 0, 0)),
        out_shape=jax.ShapeDtypeStruct((E, C, D), f32),
    )(de3, W1, B1.reshape(E, 1, FFN), W2, B2.reshape(E, 1, D))


def _combine_body(x_ref, r1_ref, r2_ref, g1_ref, g2_ref, o_ref):
    o_ref[...] = (x_ref[...] + g1_ref[...] * r1_ref[...]
                  + g2_ref[...] * r2_ref[...])


def _combine(x2, rows1, rows2, g1, g2, bm=512):
    nm = S // bm
    return pl.pallas_call(
        _combine_body,
        grid=(nm,),
        in_specs=[
            pl.BlockSpec((bm, D), lambda i: (i, 0)),
            pl.BlockSpec((bm, D), lambda i: (i, 0)),
            pl.BlockSpec((bm, D), lambda i: (i, 0)),
            pl.BlockSpec((bm, 1), lambda i: (i, 0)),
            pl.BlockSpec((bm, 1), lambda i: (i, 0)),
        ],
        out_specs=pl.BlockSpec((bm, D), lambda i: (i, 0)),
        out_shape=jax.ShapeDtypeStruct((S, D), f32),
    )(x2, rows1, rows2, g1, g2)


def kernel(x, ln1_g, ln1_b, Wq, bq, Wk, bk, Wv, bv, Wo, bo, ln2_g, ln2_b,
           Wg, W1, B1, W2, B2):
    x2d = x.reshape(S, D)
    h = _ln(x2d, ln1_g, ln1_b)
    q = _mm(h, Wq, bq)
    k = _mm(h, Wk, bk)
    v = _mm(h, Wv, bv)
    ao = _attn(q, k, v)
    x2 = _mm(ao, Wo, bo, res=x2d)
    h2 = _ln(x2, ln2_g, ln2_b)
    Wg_pad = jnp.zeros((D, EP), f32).at[:, :E].set(Wg)
    logits = _logits(h2, Wg_pad)
    g1, g2, i1, i2, s2t, laux = _gate(logits)
    de = _sc_gather(h2, s2t.reshape(E * C))
    eo = _ffn(de.reshape(E, C, D), W1, B1, W2, B2)
    eo2 = eo.reshape(E * C, D)
    rows = _sc_gather(eo2, jnp.concatenate([i1.reshape(S), i2.reshape(S)]))
    out = _combine(x2, rows[:S], rows[S:], g1, g2)
    return out.reshape(S, B, D), laux[0, 0]
